# baseline (speedup 1.0000x reference)
import functools

import jax
import jax.numpy as jnp
from jax import lax
from jax.experimental import pallas as pl
from jax.experimental.pallas import tpu as pltpu
from jax.experimental.pallas import tpu_sc as plsc

_N_EMB = 1000000
_DIM = 64
_BATCH = 16384
_LANES = 128
_NBLK = (_N_EMB + _LANES - 1) // _LANES

_info = plsc.get_sparse_core_info()
_NC, _NS = _info.num_cores, _info.num_subcores
_NW = _NC * _NS
_NPAIR = (_NBLK + 1) // 2
_PPW = (_NPAIR + _NW - 1) // _NW
_BPW = 2 * _PPW
_CAP = _BATCH
_RING = 4
_PW = 2 * _LANES
_CLAMP = _N_EMB // _LANES * _LANES - _LANES
_OCH = 64
_NCH = _CAP // _OCH
_DUMP = _BATCH

_mesh = plsc.VectorSubcoreMesh(core_axis_name="c", subcore_axis_name="s")


@functools.partial(
    pl.kernel,
    mesh=_mesh,
    compiler_params=pltpu.CompilerParams(needs_layout_passes=False),
    out_type=jax.ShapeDtypeStruct((_BATCH + _OCH, _LANES), jnp.float32),
    scratch_types=[
        pltpu.VMEM((2048,), jnp.int32),
        pltpu.VMEM((_CAP + 16,), jnp.int32),
        pltpu.VMEM((_NCH + 1, _OCH), jnp.int32),
        pltpu.VMEM((272,), jnp.int32),
        pltpu.VMEM((272,), jnp.int32),
        pltpu.VMEM((272,), jnp.int32),
        pltpu.VMEM((272,), jnp.int32),
        pltpu.VMEM((_RING, _DIM, _PW), jnp.float32),
        pltpu.VMEM((_OCH, _LANES), jnp.float32),
        pltpu.SemaphoreType.DMA,
    ],
)
def _gather_kernel(
    idx_hbm, tt_hbm, out_hbm,
    idx_buf, mkb, sb, hist, starts, offs, plist, ring, ostage, sem,
):
    wid = lax.axis_index("s") * _NC + lax.axis_index("c")
    c0 = wid * _BPW
    lane = lax.iota(jnp.int32, 16)

    def ext(ref, pos):
        vec = ref[pl.ds((pos >> 4) << 4, 16)]
        return jnp.sum(jnp.where(lane == (pos & 15), vec, 0))

    zeros16 = jnp.zeros((16,), jnp.int32)
    for t in range(16):
        hist[pl.ds(t * 16, 16)] = zeros16

    count = jnp.int32(0)
    for oc in range(_BATCH // 2048):
        pltpu.sync_copy(idx_hbm.at[pl.ds(oc * 2048, 2048)], idx_buf)

        def abody(i, cnt, oc=oc):
            vec = idx_buf[pl.ds(i * 16, 16)]
            crel = (vec >> 7) - c0
            mask = (crel >= 0) & (crel < _BPW)
            mi = mask.astype(jnp.int32)
            pos = cnt + plsc.cumsum(mi) - mi
            packed = (
                (oc * 2048 + i * 16 + lane)
                | ((vec & (_PW - 1)) << 14)
                | (crel << 22)
            )
            plsc.store_scatter(mkb, [pos], packed, mask=mask)
            plsc.addupdate_scatter(hist, [crel], mi, mask=mask)
            return cnt + jnp.sum(mi)

        count = plsc.parallel_loop(0, 2048 // 16, unroll=4, carry=count)(abody)

    carry = jnp.int32(0)
    for t in range(16):
        h = hist[pl.ds(t * 16, 16)]
        excl = plsc.cumsum(h) - h + carry
        starts[pl.ds(t * 16, 16)] = excl
        offs[pl.ds(t * 16, 16)] = excl
        carry = carry + jnp.sum(h)

    def b3(j, _):
        pj = ext(mkb, j)
        cb = jnp.broadcast_to(pj >> 22, (16,))
        o = plsc.load_gather(offs, [cb])
        plsc.store_scatter(offs, [cb], o + 1)
        plsc.store_scatter(
            sb,
            [o >> 6, o & (_OCH - 1)],
            jnp.broadcast_to(pj & ((1 << 22) - 1), (16,)),
        )
        return 0

    lax.fori_loop(0, count, b3, 0)

    nb = jnp.int32(0)
    for t in range(8):
        he = plsc.load_gather(hist, [t * 32 + 2 * lane])
        ho = plsc.load_gather(hist, [t * 32 + 2 * lane + 1])
        h = he + ho
        m = (h > 0).astype(jnp.int32)
        pos = nb + plsc.cumsum(m) - m
        posm = jnp.where(h > 0, pos, 256)
        plsc.store_scatter(plist, [posm], t * 16 + lane)
        nb = nb + jnp.sum(m)

    p0 = wid * _PPW

    def fetch(slot, bi):
        col = pl.multiple_of(
            jnp.minimum((ext(plist, bi) + p0) * _PW, _CLAMP), _LANES
        )
        pltpu.async_copy(tt_hbm.at[:, pl.ds(col, _PW)], ring.at[slot], sem)

    for s in range(_RING):

        @pl.when(s < nb)
        def _(s=s):
            fetch(s, s)

    def proc(bi, j):
        slot = bi % _RING
        pltpu.make_async_copy(
            tt_hbm.at[:, pl.ds(0, _PW)], ring.at[slot], sem
        ).wait()
        p_rel = ext(plist, bi)
        e_c = ext(starts, 2 * p_rel + 2)

        def mbody(j2, _):
            q = j2 >> 6
            ln = j2 & (_OCH - 1)
            vec = sb[q, pl.ds((ln >> 4) << 4, 16)]
            pv = jnp.sum(jnp.where(lane == (ln & 15), vec, 0))
            lvec = jnp.broadcast_to(pv >> 14, (16,))
            for v in range(4):
                vals = plsc.load_gather(ring.at[slot], [lane + 16 * v, lvec])
                ostage[ln, pl.ds(16 * v, 16)] = vals

            @pl.when(ln == _OCH - 1)
            def _():
                for g in range(4):
                    cur = sb[q, pl.ds(g * 16, 16)]
                    sb[q, pl.ds(g * 16, 16)] = cur & 16383
                pltpu.sync_copy(ostage, out_hbm.at[sb.at[q]])

            return 0

        lax.fori_loop(j, e_c, mbody, 0)

        @pl.when(bi + _RING < nb)
        def _():
            fetch(slot, bi + _RING)

        return e_c

    count_end = lax.fori_loop(0, nb, proc, jnp.int32(0))

    rem = count_end & (_OCH - 1)

    @pl.when(rem > 0)
    def _():
        q = count_end >> 6
        for g in range(4):
            cur = sb[q, pl.ds(g * 16, 16)]
            msk = (g * 16 + lane) < rem
            sb[q, pl.ds(g * 16, 16)] = jnp.where(msk, cur & 16383, _DUMP)
        pltpu.sync_copy(ostage, out_hbm.at[sb.at[q]])


@jax.jit
def kernel(x, table):
    idx = x.reshape(-1)
    out_pad = _gather_kernel(idx, table.T)
    return out_pad[:_BATCH, :_DIM]


# speedup vs baseline: 1.1014x; 1.1014x over previous
import functools

import jax
import jax.numpy as jnp
from jax import lax
from jax.experimental import pallas as pl
from jax.experimental.pallas import tpu as pltpu
from jax.experimental.pallas import tpu_sc as plsc

_N_EMB = 1000000
_DIM = 64
_BATCH = 16384
_LANES = 128
_NBLK = (_N_EMB + _LANES - 1) // _LANES

_info = plsc.get_sparse_core_info()
_NC, _NS = _info.num_cores, _info.num_subcores
_NW = _NC * _NS
_BPW = (_NBLK + _NW - 1) // _NW
_CAP = _BATCH
_RING = 8
_OCH = 64
_NCH = _CAP // _OCH
_DUMP = _BATCH

_mesh = plsc.VectorSubcoreMesh(core_axis_name="c", subcore_axis_name="s")


@functools.partial(
    pl.kernel,
    mesh=_mesh,
    compiler_params=pltpu.CompilerParams(needs_layout_passes=False),
    out_type=jax.ShapeDtypeStruct((_BATCH + _OCH, _LANES), jnp.float32),
    scratch_types=[
        pltpu.VMEM((2048,), jnp.int32),
        pltpu.VMEM((_CAP + 16,), jnp.int32),
        pltpu.VMEM((_NCH + 1, _OCH), jnp.int32),
        pltpu.VMEM((272,), jnp.int32),
        pltpu.VMEM((272,), jnp.int32),
        pltpu.VMEM((272,), jnp.int32),
        pltpu.VMEM((272,), jnp.int32),
        pltpu.VMEM((32,), jnp.int32),
        pltpu.VMEM((_RING, _DIM, _LANES), jnp.float32),
        pltpu.VMEM((_OCH, _LANES), jnp.float32),
        pltpu.SemaphoreType.DMA,
    ],
)
def _gather_kernel(
    idx_hbm, tt_hbm, out_hbm,
    idx_buf, mkb, sb, hist, starts, offs, blist, tmp, ring, ostage, sem,
):
    wid = lax.axis_index("s") * _NC + lax.axis_index("c")
    c0 = wid * _BPW
    lane = lax.iota(jnp.int32, 16)

    def ext(ref, pos):
        vec = ref[pl.ds((pos >> 4) << 4, 16)]
        return jnp.sum(jnp.where(lane == (pos & 15), vec, 0))

    zeros16 = jnp.zeros((16,), jnp.int32)
    for t in range(16):
        hist[pl.ds(t * 16, 16)] = zeros16

    count = jnp.int32(0)
    for oc in range(_BATCH // 2048):
        pltpu.sync_copy(idx_hbm.at[pl.ds(oc * 2048, 2048)], idx_buf)

        def abody(i, cnt, oc=oc):
            vec = idx_buf[pl.ds(i * 16, 16)]
            crel = (vec >> 7) - c0
            mask = (crel >= 0) & (crel < _BPW)
            mi = mask.astype(jnp.int32)
            pos = cnt + plsc.cumsum(mi) - mi
            packed = (
                (oc * 2048 + i * 16 + lane)
                | ((vec & (_LANES - 1)) << 14)
                | (crel << 21)
            )
            plsc.store_scatter(mkb, [pos], packed, mask=mask)
            plsc.addupdate_scatter(hist, [crel], mi, mask=mask)
            return cnt + jnp.sum(mi)

        count = plsc.parallel_loop(0, 2048 // 16, unroll=4, carry=count)(abody)

    carry = jnp.int32(0)
    for t in range(16):
        h = hist[pl.ds(t * 16, 16)]
        excl = plsc.cumsum(h) - h + carry
        starts[pl.ds(t * 16, 16)] = excl
        offs[pl.ds(t * 16, 16)] = excl
        carry = carry + jnp.sum(h)

    plsc.store_scatter(
        mkb, [count + lane], jnp.broadcast_to(jnp.int32(250 << 21), (16,))
    )
    tmp[pl.ds(0, 16)] = jnp.where(lane == 0, -1, 0)
    tmp[pl.ds(16, 16)] = jnp.where(lane == 1, -2, 0)

    def b3(g, _):
        kv = mkb[pl.ds(g * 16, 16)]
        sk2, _sv = plsc.sort_key_val(kv, kv)
        cb = sk2 >> 21
        plsc.store_scatter(tmp, [lane + 1], cb)
        prev = tmp[pl.ds(0, 16)]
        nxt = plsc.load_gather(tmp, [lane + 2])
        head = cb != prev
        rank = lane - plsc.cummax(jnp.where(head, lane, 0))
        o = plsc.load_gather(offs, [cb]) + rank
        plsc.store_scatter(offs, [cb], o + 1, mask=cb != nxt)
        plsc.store_scatter(sb, [o >> 6, o & (_OCH - 1)], sk2 & ((1 << 21) - 1))
        return 0

    lax.fori_loop(0, (count + 15) >> 4, b3, 0)

    nb = jnp.int32(0)
    for t in range(16):
        h = hist[pl.ds(t * 16, 16)]
        m = (h > 0).astype(jnp.int32)
        pos = nb + plsc.cumsum(m) - m
        posm = jnp.where(h > 0, pos, 256)
        plsc.store_scatter(blist, [posm], t * 16 + lane)
        nb = nb + jnp.sum(m)

    def fetch(slot, bi):
        col = pl.multiple_of((ext(blist, bi) + c0) * _LANES, _LANES)
        pltpu.async_copy(tt_hbm.at[:, pl.ds(col, _LANES)], ring.at[slot], sem)

    for s in range(_RING):

        @pl.when(s < nb)
        def _(s=s):
            fetch(s, s)

    def proc(bi, j):
        slot = bi % _RING
        pltpu.make_async_copy(
            tt_hbm.at[:, pl.ds(0, _LANES)], ring.at[slot], sem
        ).wait()
        c_rel = ext(blist, bi)
        e_c = ext(starts, c_rel + 1)

        def mbody(j2, _):
            q = j2 >> 6
            ln = j2 & (_OCH - 1)
            vec = sb[q, pl.ds((ln >> 4) << 4, 16)]
            pv = jnp.sum(jnp.where(lane == (ln & 15), vec, 0))
            lvec = jnp.broadcast_to(pv >> 14, (16,))
            for v in range(4):
                vals = plsc.load_gather(ring.at[slot], [lane + 16 * v, lvec])
                ostage[ln, pl.ds(16 * v, 16)] = vals

            @pl.when(ln == _OCH - 1)
            def _():
                for g in range(4):
                    cur = sb[q, pl.ds(g * 16, 16)]
                    sb[q, pl.ds(g * 16, 16)] = cur & 16383
                pltpu.sync_copy(ostage, out_hbm.at[sb.at[q]])

            return 0

        lax.fori_loop(j, e_c, mbody, 0)

        @pl.when(bi + _RING < nb)
        def _():
            fetch(slot, bi + _RING)

        return e_c

    count_end = lax.fori_loop(0, nb, proc, jnp.int32(0))

    rem = count_end & (_OCH - 1)

    @pl.when(rem > 0)
    def _():
        q = count_end >> 6
        for g in range(4):
            cur = sb[q, pl.ds(g * 16, 16)]
            msk = (g * 16 + lane) < rem
            sb[q, pl.ds(g * 16, 16)] = jnp.where(msk, cur & 16383, _DUMP)
        pltpu.sync_copy(ostage, out_hbm.at[sb.at[q]])


@jax.jit
def kernel(x, table):
    idx = x.reshape(-1)
    out_pad = _gather_kernel(idx, table.T)
    return out_pad[:_BATCH, :_DIM]


# speedup vs baseline: 1.1189x; 1.0160x over previous
import functools

import jax
import jax.numpy as jnp
from jax import lax
from jax.experimental import pallas as pl
from jax.experimental.pallas import tpu as pltpu
from jax.experimental.pallas import tpu_sc as plsc

_N_EMB = 1000000
_DIM = 64
_BATCH = 16384
_LANES = 128
_NBLK = (_N_EMB + _LANES - 1) // _LANES

_info = plsc.get_sparse_core_info()
_NC, _NS = _info.num_cores, _info.num_subcores
_NW = _NC * _NS
_BPW = (_NBLK + _NW - 1) // _NW
_CAP = _BATCH
_RING = 8
_OCH = 64
_NCH = _CAP // _OCH
_DUMP = _BATCH

_mesh = plsc.VectorSubcoreMesh(core_axis_name="c", subcore_axis_name="s")


@functools.partial(
    pl.kernel,
    mesh=_mesh,
    compiler_params=pltpu.CompilerParams(needs_layout_passes=False),
    out_type=jax.ShapeDtypeStruct((_BATCH + _OCH, _LANES), jnp.float32),
    scratch_types=[
        pltpu.VMEM((2, 2048), jnp.int32),
        pltpu.VMEM((_CAP + 16,), jnp.int32),
        pltpu.VMEM((_NCH + 1, _OCH), jnp.int32),
        pltpu.VMEM((272,), jnp.int32),
        pltpu.VMEM((272,), jnp.int32),
        pltpu.VMEM((272,), jnp.int32),
        pltpu.VMEM((272,), jnp.int32),
        pltpu.VMEM((32,), jnp.int32),
        pltpu.VMEM((_RING, _DIM, _LANES), jnp.float32),
        pltpu.VMEM((_OCH, _LANES), jnp.float32),
        pltpu.SemaphoreType.DMA,
        pltpu.SemaphoreType.DMA,
        pltpu.SemaphoreType.DMA,
    ],
)
def _gather_kernel(
    idx_hbm, tt_hbm, out_hbm,
    idx_buf, mkb, sb, hist, starts, offs, blist, tmp, ring, ostage,
    sem, isem0, isem1,
):
    wid = lax.axis_index("s") * _NC + lax.axis_index("c")
    c0 = wid * _BPW
    lane = lax.iota(jnp.int32, 16)

    def ext(ref, pos):
        vec = ref[pl.ds((pos >> 4) << 4, 16)]
        return jnp.sum(jnp.where(lane == (pos & 15), vec, 0))

    isems = (isem0, isem1)
    pltpu.async_copy(idx_hbm.at[pl.ds(0, 2048)], idx_buf.at[0], isems[0])

    zeros16 = jnp.zeros((16,), jnp.int32)
    for t in range(16):
        hist[pl.ds(t * 16, 16)] = zeros16

    count = jnp.int32(0)
    for oc in range(_BATCH // 2048):
        pltpu.make_async_copy(
            idx_hbm.at[pl.ds(0, 2048)], idx_buf.at[oc & 1], isems[oc & 1]
        ).wait()
        if oc + 1 < _BATCH // 2048:
            pltpu.async_copy(
                idx_hbm.at[pl.ds((oc + 1) * 2048, 2048)],
                idx_buf.at[(oc + 1) & 1],
                isems[(oc + 1) & 1],
            )

        def abody(i, cnt, oc=oc):
            vec = idx_buf[oc & 1, pl.ds(i * 16, 16)]
            crel = (vec >> 7) - c0
            mask = (crel >= 0) & (crel < _BPW)
            mi = mask.astype(jnp.int32)
            pos = cnt + plsc.cumsum(mi) - mi
            packed = (
                (oc * 2048 + i * 16 + lane)
                | ((vec & (_LANES - 1)) << 14)
                | (crel << 21)
            )
            plsc.store_scatter(mkb, [pos], packed, mask=mask)
            plsc.addupdate_scatter(hist, [crel], mi, mask=mask)
            return cnt + jnp.sum(mi)

        count = plsc.parallel_loop(0, 2048 // 16, unroll=8, carry=count)(abody)

    carry = jnp.int32(0)
    for t in range(16):
        h = hist[pl.ds(t * 16, 16)]
        excl = plsc.cumsum(h) - h + carry
        starts[pl.ds(t * 16, 16)] = excl
        offs[pl.ds(t * 16, 16)] = excl
        carry = carry + jnp.sum(h)

    plsc.store_scatter(
        mkb, [count + lane], jnp.broadcast_to(jnp.int32(250 << 21), (16,))
    )
    tmp[pl.ds(0, 16)] = jnp.where(lane == 0, -1, 0)
    tmp[pl.ds(16, 16)] = jnp.where(lane == 1, -2, 0)

    def b3(g, _):
        kv = mkb[pl.ds(g * 16, 16)]
        sk2, _sv = plsc.sort_key_val(kv, kv)
        cb = sk2 >> 21
        plsc.store_scatter(tmp, [lane + 1], cb)
        prev = tmp[pl.ds(0, 16)]
        nxt = plsc.load_gather(tmp, [lane + 2])
        head = cb != prev
        rank = lane - plsc.cummax(jnp.where(head, lane, 0))
        o = plsc.load_gather(offs, [cb]) + rank
        plsc.store_scatter(offs, [cb], o + 1, mask=cb != nxt)
        plsc.store_scatter(sb, [o >> 6, o & (_OCH - 1)], sk2 & ((1 << 21) - 1))
        return 0

    lax.fori_loop(0, (count + 15) >> 4, b3, 0)

    nb = jnp.int32(0)
    for t in range(16):
        h = hist[pl.ds(t * 16, 16)]
        m = (h > 0).astype(jnp.int32)
        pos = nb + plsc.cumsum(m) - m
        posm = jnp.where(h > 0, pos, 256)
        plsc.store_scatter(blist, [posm], t * 16 + lane)
        nb = nb + jnp.sum(m)

    def fetch(slot, bi):
        col = pl.multiple_of((ext(blist, bi) + c0) * _LANES, _LANES)
        pltpu.async_copy(tt_hbm.at[:, pl.ds(col, _LANES)], ring.at[slot], sem)

    for s in range(_RING):

        @pl.when(s < nb)
        def _(s=s):
            fetch(s, s)

    def proc(bi, j):
        slot = bi % _RING
        pltpu.make_async_copy(
            tt_hbm.at[:, pl.ds(0, _LANES)], ring.at[slot], sem
        ).wait()
        c_rel = ext(blist, bi)
        e_c = ext(starts, c_rel + 1)

        def mbody(j2, _):
            q = j2 >> 6
            ln = j2 & (_OCH - 1)
            vec = sb[q, pl.ds((ln >> 4) << 4, 16)]
            pv = jnp.sum(jnp.where(lane == (ln & 15), vec, 0))
            lvec = jnp.broadcast_to(pv >> 14, (16,))
            for v in range(4):
                vals = plsc.load_gather(ring.at[slot], [lane + 16 * v, lvec])
                ostage[ln, pl.ds(16 * v, 16)] = vals

            @pl.when(ln == _OCH - 1)
            def _():
                for g in range(4):
                    cur = sb[q, pl.ds(g * 16, 16)]
                    sb[q, pl.ds(g * 16, 16)] = cur & 16383
                pltpu.sync_copy(ostage, out_hbm.at[sb.at[q]])

            return 0

        lax.fori_loop(j, e_c, mbody, 0)

        @pl.when(bi + _RING < nb)
        def _():
            fetch(slot, bi + _RING)

        return e_c

    count_end = lax.fori_loop(0, nb, proc, jnp.int32(0))

    rem = count_end & (_OCH - 1)

    @pl.when(rem > 0)
    def _():
        q = count_end >> 6
        for g in range(4):
            cur = sb[q, pl.ds(g * 16, 16)]
            msk = (g * 16 + lane) < rem
            sb[q, pl.ds(g * 16, 16)] = jnp.where(msk, cur & 16383, _DUMP)
        pltpu.sync_copy(ostage, out_hbm.at[sb.at[q]])


@jax.jit
def kernel(x, table):
    idx = x.reshape(-1)
    out_pad = _gather_kernel(idx, table.T)
    return out_pad[:_BATCH, :_DIM]


# speedup vs baseline: 1.1227x; 1.0033x over previous
import functools

import jax
import jax.numpy as jnp
from jax import lax
from jax.experimental import pallas as pl
from jax.experimental.pallas import tpu as pltpu
from jax.experimental.pallas import tpu_sc as plsc

_N_EMB = 1000000
_DIM = 64
_BATCH = 16384
_LANES = 128
_NBLK = (_N_EMB + _LANES - 1) // _LANES

_info = plsc.get_sparse_core_info()
_NC, _NS = _info.num_cores, _info.num_subcores
_NW = _NC * _NS
_BPW = (_NBLK + _NW - 1) // _NW
_CAP = _BATCH
_RING = 8
_OCH = 64
_NCH = _CAP // _OCH
_DUMP = _BATCH

_mesh = plsc.VectorSubcoreMesh(core_axis_name="c", subcore_axis_name="s")


@functools.partial(
    pl.kernel,
    mesh=_mesh,
    compiler_params=pltpu.CompilerParams(needs_layout_passes=False),
    out_type=jax.ShapeDtypeStruct((_BATCH + _OCH, _LANES), jnp.float32),
    scratch_types=[
        pltpu.VMEM((2, 2048), jnp.int32),
        pltpu.VMEM((_CAP + 16,), jnp.int32),
        pltpu.VMEM((_NCH + 1, _OCH), jnp.int32),
        pltpu.VMEM((272,), jnp.int32),
        pltpu.VMEM((272,), jnp.int32),
        pltpu.VMEM((272,), jnp.int32),
        pltpu.VMEM((272,), jnp.int32),
        pltpu.VMEM((32,), jnp.int32),
        pltpu.VMEM((_RING, _DIM, _LANES), jnp.float32),
        pltpu.VMEM((_OCH, _LANES), jnp.float32),
        pltpu.SemaphoreType.DMA,
        pltpu.SemaphoreType.DMA,
        pltpu.SemaphoreType.DMA,
    ],
)
def _gather_kernel(
    idx_hbm, tt_hbm, out_hbm,
    idx_buf, mkb, sb, hist, starts, offs, blist, tmp, ring, ostage,
    sem, isem0, isem1,
):
    wid = lax.axis_index("s") * _NC + lax.axis_index("c")
    c0 = wid * _BPW
    lane = lax.iota(jnp.int32, 16)

    def ext(ref, pos):
        vec = ref[pl.ds((pos >> 4) << 4, 16)]
        return jnp.sum(jnp.where(lane == (pos & 15), vec, 0))

    isems = (isem0, isem1)
    pltpu.async_copy(idx_hbm.at[pl.ds(0, 2048)], idx_buf.at[0], isems[0])

    zeros16 = jnp.zeros((16,), jnp.int32)
    for t in range(16):
        hist[pl.ds(t * 16, 16)] = zeros16

    count = jnp.int32(0)
    for oc in range(_BATCH // 2048):
        pltpu.make_async_copy(
            idx_hbm.at[pl.ds(0, 2048)], idx_buf.at[oc & 1], isems[oc & 1]
        ).wait()
        if oc + 1 < _BATCH // 2048:
            pltpu.async_copy(
                idx_hbm.at[pl.ds((oc + 1) * 2048, 2048)],
                idx_buf.at[(oc + 1) & 1],
                isems[(oc + 1) & 1],
            )

        def abody(i, cnt, oc=oc):
            vec = idx_buf[oc & 1, pl.ds(i * 16, 16)]
            crel = (vec >> 7) - c0
            mask = (crel >= 0) & (crel < _BPW)
            mi = mask.astype(jnp.int32)
            pos = cnt + plsc.cumsum(mi) - mi
            packed = (
                (oc * 2048 + i * 16 + lane)
                | ((vec & (_LANES - 1)) << 14)
                | (crel << 21)
            )
            plsc.store_scatter(mkb, [pos], packed, mask=mask)
            plsc.addupdate_scatter(hist, [crel], mi, mask=mask)
            return cnt + jnp.sum(mi)

        count = plsc.parallel_loop(0, 2048 // 16, unroll=8, carry=count)(abody)

    carry = jnp.int32(0)
    for t in range(16):
        h = hist[pl.ds(t * 16, 16)]
        excl = plsc.cumsum(h) - h + carry
        starts[pl.ds(t * 16, 16)] = excl
        offs[pl.ds(t * 16, 16)] = excl
        carry = carry + jnp.sum(h)

    nb = jnp.int32(0)
    for t in range(16):
        h = hist[pl.ds(t * 16, 16)]
        m = (h > 0).astype(jnp.int32)
        pos = nb + plsc.cumsum(m) - m
        posm = jnp.where(h > 0, pos, 256)
        plsc.store_scatter(blist, [posm], t * 16 + lane)
        nb = nb + jnp.sum(m)

    def fetch(slot, bi):
        col = pl.multiple_of((ext(blist, bi) + c0) * _LANES, _LANES)
        pltpu.async_copy(tt_hbm.at[:, pl.ds(col, _LANES)], ring.at[slot], sem)

    for s in range(_RING):

        @pl.when(s < nb)
        def _(s=s):
            fetch(s, s)

    plsc.store_scatter(
        mkb, [count + lane], jnp.broadcast_to(jnp.int32(250 << 21), (16,))
    )
    tmp[pl.ds(0, 16)] = jnp.where(lane == 0, -1, 0)
    tmp[pl.ds(16, 16)] = jnp.where(lane == 1, -2, 0)

    def b3(g, _):
        kv = mkb[pl.ds(g * 16, 16)]
        sk2, _sv = plsc.sort_key_val(kv, kv)
        cb = sk2 >> 21
        plsc.store_scatter(tmp, [lane + 1], cb)
        prev = tmp[pl.ds(0, 16)]
        nxt = plsc.load_gather(tmp, [lane + 2])
        head = cb != prev
        rank = lane - plsc.cummax(jnp.where(head, lane, 0))
        o = plsc.load_gather(offs, [cb]) + rank
        plsc.store_scatter(offs, [cb], o + 1, mask=cb != nxt)
        plsc.store_scatter(sb, [o >> 6, o & (_OCH - 1)], sk2 & ((1 << 21) - 1))
        return 0

    lax.fori_loop(0, (count + 15) >> 4, b3, 0)

    def proc(bi, j):
        slot = bi % _RING
        pltpu.make_async_copy(
            tt_hbm.at[:, pl.ds(0, _LANES)], ring.at[slot], sem
        ).wait()
        c_rel = ext(blist, bi)
        e_c = ext(starts, c_rel + 1)

        def mbody(j2, _):
            q = j2 >> 6
            ln = j2 & (_OCH - 1)
            vec = sb[q, pl.ds((ln >> 4) << 4, 16)]
            pv = jnp.sum(jnp.where(lane == (ln & 15), vec, 0))
            lvec = jnp.broadcast_to(pv >> 14, (16,))
            for v in range(4):
                vals = plsc.load_gather(ring.at[slot], [lane + 16 * v, lvec])
                ostage[ln, pl.ds(16 * v, 16)] = vals

            @pl.when(ln == _OCH - 1)
            def _():
                for g in range(4):
                    cur = sb[q, pl.ds(g * 16, 16)]
                    sb[q, pl.ds(g * 16, 16)] = cur & 16383
                pltpu.sync_copy(ostage, out_hbm.at[sb.at[q]])

            return 0

        lax.fori_loop(j, e_c, mbody, 0)

        @pl.when(bi + _RING < nb)
        def _():
            fetch(slot, bi + _RING)

        return e_c

    count_end = lax.fori_loop(0, nb, proc, jnp.int32(0))

    rem = count_end & (_OCH - 1)

    @pl.when(rem > 0)
    def _():
        q = count_end >> 6
        for g in range(4):
            cur = sb[q, pl.ds(g * 16, 16)]
            msk = (g * 16 + lane) < rem
            sb[q, pl.ds(g * 16, 16)] = jnp.where(msk, cur & 16383, _DUMP)
        pltpu.sync_copy(ostage, out_hbm.at[sb.at[q]])


@jax.jit
def kernel(x, table):
    idx = x.reshape(-1)
    out_pad = _gather_kernel(idx, table.T)
    return out_pad[:_BATCH, :_DIM]
